# baseline (speedup 1.0000x reference)
import functools

import jax
import jax.numpy as jnp
from jax import lax
from jax.experimental import pallas as pl
from jax.experimental.pallas import tpu as pltpu
from jax.experimental.pallas import tpu_sc as plsc

_B, _N = 8, 20000
_PALL, _NPT = 1200, 1024
_D, _H, _NL, _DFF, _DOUT = 96, 6, 2, 384, 288
_DH = _D // _H
_MAXN = 34
_RADIUS = 2.5 * 0.04 * 1.0
_CHUNKS = 157
_NPAD = _CHUNKS * 128
_T = _B * _NPT


def _fps_body(xyz_ref, lin_ref, out_ref, dists_ref):
    lin3 = (lax.broadcasted_iota(jnp.int32, (_CHUNKS, _B, 128), 0) * 128
            + lax.broadcasted_iota(jnp.int32, (_CHUNKS, _B, 128), 2)
            ).astype(jnp.float32)
    dists_ref[...] = jnp.where(lin3 < float(_N), 1e10, -1.0)
    x0 = xyz_ref[0, 0, :, 0:1]
    y0 = xyz_ref[1, 0, :, 0:1]
    z0 = xyz_ref[2, 0, :, 0:1]

    ngroups = 3
    bounds = [(_CHUNKS * g) // ngroups for g in range(ngroups + 1)]

    def body(i, state):
        farf, fx, fy, fz = state
        out_ref[pl.ds(i, 1), :] = farf.reshape(1, _B)
        st = []
        for g in range(ngroups):
            gs = None
            for c in range(bounds[g], bounds[g + 1]):
                xc = xyz_ref[0, c]
                yc = xyz_ref[1, c]
                zc = xyz_ref[2, c]
                dxc = xc - fx
                dyc = yc - fy
                dzc = zc - fz
                dc = dxc * dxc + dyc * dyc + dzc * dzc
                ndc = jnp.minimum(dists_ref[c], dc)
                dists_ref[c] = ndc
                pc_ = lin_ref[c]
                if gs is None:
                    gs = (ndc, pc_, xc, yc, zc)
                else:
                    upd = ndc > gs[0]
                    gs = tuple(jnp.where(upd, new, old) for new, old in
                               zip((ndc, pc_, xc, yc, zc), gs))
            st.append(gs)
        while len(st) > 1:
            nxt = []
            for a in range(0, len(st) - 1, 2):
                lo, hi = st[a], st[a + 1]
                upd = hi[0] > lo[0]
                nxt.append(tuple(jnp.where(upd, h, l)
                                 for h, l in zip(hi, lo)))
            if len(st) % 2:
                nxt.append(st[-1])
            st = nxt
        runmax, pbest, xb, yb, zb = st[0]
        m = jnp.max(runmax, axis=-1, keepdims=True)
        plin = jnp.where(runmax == m, pbest, 1e9)
        nfar = jnp.min(plin, axis=-1, keepdims=True)
        fsel = plin == nfar
        nfx = jnp.sum(jnp.where(fsel, xb, 0.0), axis=-1, keepdims=True)
        nfy = jnp.sum(jnp.where(fsel, yb, 0.0), axis=-1, keepdims=True)
        nfz = jnp.sum(jnp.where(fsel, zb, 0.0), axis=-1, keepdims=True)
        return (nfar, nfx, nfy, nfz)

    lax.fori_loop(0, _PALL, body,
                  (jnp.zeros((_B, 1), jnp.float32), x0, y0, z0))


def _run_fps(point_clouds):
    pcs = jnp.transpose(point_clouds, (2, 0, 1))
    pcs = jnp.pad(pcs, ((0, 0), (0, 0), (0, _NPAD - _N)))
    xyz_t = pcs.reshape(3, _B, _CHUNKS, 128).transpose(0, 2, 1, 3)
    lin = (jnp.arange(_CHUNKS, dtype=jnp.int32)[:, None, None] * 128
           + jnp.arange(128, dtype=jnp.int32)[None, None, :]
           + jnp.zeros((1, _B, 1), jnp.int32)).astype(jnp.float32)
    return pl.pallas_call(
        _fps_body,
        out_shape=jax.ShapeDtypeStruct((_PALL, _B), jnp.float32),
        scratch_shapes=[pltpu.VMEM((_CHUNKS, _B, 128), jnp.float32)],
    )(xyz_t, lin)


_SC_NW = 32
_SC_ROWS = _T // _SC_NW
_SC_CHUNK = 128


def _sc_gather_body(tab_ref, idx_ref, out_ref, idx_v, rows_v, sem):
    wid = lax.axis_index("s") * 2 + lax.axis_index("c")
    base = wid * _SC_ROWS
    for j in range(_SC_ROWS // _SC_CHUNK):
        off = base + j * _SC_CHUNK
        pltpu.sync_copy(idx_ref.at[pl.ds(off, _SC_CHUNK)], idx_v)
        pltpu.async_copy(tab_ref.at[idx_v], rows_v, sem).wait()
        pltpu.sync_copy(rows_v, out_ref.at[pl.ds(off, _SC_CHUNK)])


def _run_sc_gather(table, flat_idx):
    mesh = plsc.VectorSubcoreMesh(core_axis_name="c", subcore_axis_name="s")
    k = functools.partial(
        pl.kernel,
        mesh=mesh,
        out_type=jax.ShapeDtypeStruct((_T, 16), jnp.float32),
        scratch_types=[
            pltpu.VMEM((_SC_CHUNK,), jnp.int32),
            pltpu.VMEM((_SC_CHUNK, 16), jnp.float32),
            pltpu.SemaphoreType.DMA,
        ],
        compiler_params=pltpu.CompilerParams(use_tc_tiling_on_sc=False),
    )(_sc_gather_body)
    return k(table, flat_idx)


def _layernorm(x):
    mu = jnp.mean(x, axis=-1, keepdims=True)
    xc = x - mu
    var = jnp.mean(xc * xc, axis=-1, keepdims=True)
    return xc * lax.rsqrt(var + 1e-5)


def _tf_body(feat_ref, pct_ref, win_ref, wqkv_ref, wo_ref,
             w1_ref, w2_ref, wout_ref, out_ref, c_ref, work_ref):
    feat = feat_ref[0]
    d2 = None
    for c in range(3):
        col = feat[:, 3 + c:4 + c]
        row = pct_ref[0, c:c + 1, :]
        dd = col - row
        dd = dd * dd
        d2 = dd if d2 is None else d2 + dd
    r2 = jnp.float32(_RADIUS ** 2)
    inrad = d2 <= r2
    inradf = inrad.astype(jnp.float32)
    count = jnp.sum(inradf, axis=1, keepdims=True)
    maxc = jnp.max(count)
    colidx_i = lax.broadcasted_iota(jnp.int32, (_NPT, _NPT), 1)
    rowidx_i = lax.broadcasted_iota(jnp.int32, (_NPT, _NPT), 0)
    colidx = colidx_i.astype(jnp.float32)
    diag = colidx_i == rowidx_i
    c_ref[...] = inradf + jnp.where(diag, jnp.float32(_MAXN) - count, 0.0)

    @pl.when(maxc > jnp.float32(_MAXN))
    def _rare():
        work_ref[...] = jnp.where(inrad, d2, jnp.inf)
        c_ref[...] = jnp.zeros((_NPT, _NPT), jnp.float32)
        selfcol = lax.broadcasted_iota(jnp.int32, (_NPT, 1), 0).astype(
            jnp.float32)

        def ex(j, carry):
            work = work_ref[...]
            m = jnp.min(work, axis=1, keepdims=True)
            am = jnp.min(jnp.where(work == m, colidx, 2e9), axis=1,
                         keepdims=True)
            isfin = m != jnp.inf
            chosen = jnp.where(isfin, am, selfcol)
            c_ref[...] = c_ref[...] + (colidx == chosen).astype(jnp.float32)
            work_ref[...] = jnp.where((colidx == am) & isfin, jnp.inf, work)
            return carry

        lax.fori_loop(0, _MAXN, ex, 0)

    x = jnp.dot(feat, win_ref[...], preferred_element_type=jnp.float32)
    logc = jnp.log(c_ref[...])
    inv_scale = jnp.float32(1.0 / 4.0)
    for l in range(_NL):
        xn = _layernorm(x)
        o_heads = []
        for h in range(_H):
            qkv = jnp.dot(xn, wqkv_ref[l, h],
                          preferred_element_type=jnp.float32)
            qh = qkv[:, 0:_DH]
            kh = qkv[:, _DH:2 * _DH]
            vh1 = jnp.concatenate(
                [qkv[:, 2 * _DH:3 * _DH],
                 jnp.ones((_NPT, 1), jnp.float32)], axis=1)
            logits = lax.dot_general(
                qh, kh, (((1,), (1,)), ((), ())),
                preferred_element_type=jnp.float32) * inv_scale
            neg = logits + logc
            mx = jnp.max(neg, axis=1, keepdims=True)
            e = jnp.exp(neg - mx)
            ov = jnp.dot(e, vh1, preferred_element_type=jnp.float32)
            oh = ov[:, :_DH] / ov[:, _DH:_DH + 1]
            o_heads.append(oh)
        o = jnp.concatenate(o_heads, axis=1)
        x = x + jnp.dot(o, wo_ref[l], preferred_element_type=jnp.float32)
        xn2 = _layernorm(x)
        hid = jnp.maximum(
            jnp.dot(xn2, w1_ref[l], preferred_element_type=jnp.float32), 0.0)
        x = x + jnp.dot(hid, w2_ref[l], preferred_element_type=jnp.float32)
    out = jnp.dot(_layernorm(x), wout_ref[...],
                  preferred_element_type=jnp.float32)
    out_ref[0] = out


def _run_transformer(feat, pct, win_p, wqkv, wo, w1, w2, wout):
    full = lambda *shape: pl.BlockSpec(shape, lambda b: (0,) * len(shape))
    return pl.pallas_call(
        _tf_body,
        grid=(_B,),
        in_specs=[
            pl.BlockSpec((1, _NPT, 8), lambda b: (b, 0, 0)),
            pl.BlockSpec((1, 3, _NPT), lambda b: (b, 0, 0)),
            full(8, _D),
            full(_NL, _H, _D, 3 * _DH),
            full(_NL, _D, _D),
            full(_NL, _D, _DFF),
            full(_NL, _DFF, _D),
            full(_D, _DOUT),
        ],
        out_specs=pl.BlockSpec((1, _NPT, _DOUT), lambda b: (b, 0, 0)),
        out_shape=jax.ShapeDtypeStruct((_B, _NPT, _DOUT), jnp.float32),
        scratch_shapes=[pltpu.VMEM((_NPT, _NPT), jnp.float32),
                        pltpu.VMEM((_NPT, _NPT), jnp.float32)],
    )(feat, pct, win_p, wqkv, wo, w1, w2, wout)


def kernel(point_clouds, cloud_colors, W_in, Wq, Wk, Wv, Wo, W1, W2, W_out):
    idxs_f = _run_fps(point_clouds)
    idxs = idxs_f.astype(jnp.int32)
    sel = jax.random.permutation(jax.random.key(1), _PALL)[:_NPT]
    fps_idx = idxs[sel, :].T
    flat_idx = (fps_idx
                + (jnp.arange(_B, dtype=jnp.int32) * _N)[:, None]
                ).reshape(_T)
    table = jnp.pad(jnp.concatenate([cloud_colors, point_clouds], axis=-1),
                    ((0, 0), (0, 0), (0, 10))).reshape(_B * _N, 16)
    rows = _run_sc_gather(table, flat_idx)
    feat = rows[:, :8].reshape(_B, _NPT, 8)
    pc = rows[:, 3:6].reshape(_B, _NPT, 3)
    pct = jnp.transpose(pc, (0, 2, 1))
    win_p = jnp.pad(W_in, ((0, 2), (0, 0)))
    wqkv = jnp.concatenate(
        [w.reshape(_NL, _D, _H, _DH).transpose(0, 2, 1, 3)
         for w in (Wq, Wk, Wv)], axis=-1)
    out = _run_transformer(feat, pct, win_p, wqkv, Wo, W1, W2, W_out)
    features = jnp.transpose(out, (0, 2, 1))
    return features, pc


# speedup vs baseline: 1.0103x; 1.0103x over previous
import functools

import jax
import jax.numpy as jnp
from jax import lax
from jax.experimental import pallas as pl
from jax.experimental.pallas import tpu as pltpu
from jax.experimental.pallas import tpu_sc as plsc

_B, _N = 8, 20000
_PALL, _NPT = 1200, 1024
_D, _H, _NL, _DFF, _DOUT = 96, 6, 2, 384, 288
_DH = _D // _H
_MAXN = 34
_RADIUS = 2.5 * 0.04 * 1.0
_CHUNKS = 157
_NPAD = _CHUNKS * 128
_T = _B * _NPT


def _fps_body(xyz_ref, lin_ref, out_ref, dists_ref):
    lin3 = (lax.broadcasted_iota(jnp.int32, (_CHUNKS, _B, 128), 0) * 128
            + lax.broadcasted_iota(jnp.int32, (_CHUNKS, _B, 128), 2)
            ).astype(jnp.float32)
    dists_ref[...] = jnp.where(lin3 < float(_N), 1e10, -1.0)
    x0 = xyz_ref[0, 0, :, 0:1]
    y0 = xyz_ref[1, 0, :, 0:1]
    z0 = xyz_ref[2, 0, :, 0:1]

    ngroups = 2
    bounds = [(_CHUNKS * g) // ngroups for g in range(ngroups + 1)]

    def body(i, state):
        farf, fx, fy, fz = state
        out_ref[pl.ds(i, 1), :] = farf.reshape(1, _B)
        st = []
        for g in range(ngroups):
            gs = None
            for c in range(bounds[g], bounds[g + 1]):
                xc = xyz_ref[0, c]
                yc = xyz_ref[1, c]
                zc = xyz_ref[2, c]
                dxc = xc - fx
                dyc = yc - fy
                dzc = zc - fz
                dc = dxc * dxc + dyc * dyc + dzc * dzc
                ndc = jnp.minimum(dists_ref[c], dc)
                dists_ref[c] = ndc
                pc_ = lin_ref[c]
                if gs is None:
                    gs = (ndc, pc_, xc, yc, zc)
                else:
                    upd = ndc > gs[0]
                    gs = tuple(jnp.where(upd, new, old) for new, old in
                               zip((ndc, pc_, xc, yc, zc), gs))
            st.append(gs)
        while len(st) > 1:
            nxt = []
            for a in range(0, len(st) - 1, 2):
                lo, hi = st[a], st[a + 1]
                upd = hi[0] > lo[0]
                nxt.append(tuple(jnp.where(upd, h, l)
                                 for h, l in zip(hi, lo)))
            if len(st) % 2:
                nxt.append(st[-1])
            st = nxt
        runmax, pbest, xb, yb, zb = st[0]
        m = jnp.max(runmax, axis=-1, keepdims=True)
        plin = jnp.where(runmax == m, pbest, 1e9)
        nfar = jnp.min(plin, axis=-1, keepdims=True)
        fsel = plin == nfar
        nfx = jnp.sum(jnp.where(fsel, xb, 0.0), axis=-1, keepdims=True)
        nfy = jnp.sum(jnp.where(fsel, yb, 0.0), axis=-1, keepdims=True)
        nfz = jnp.sum(jnp.where(fsel, zb, 0.0), axis=-1, keepdims=True)
        return (nfar, nfx, nfy, nfz)

    lax.fori_loop(0, _PALL, body,
                  (jnp.zeros((_B, 1), jnp.float32), x0, y0, z0))


def _run_fps(point_clouds):
    pcs = jnp.transpose(point_clouds, (2, 0, 1))
    pcs = jnp.pad(pcs, ((0, 0), (0, 0), (0, _NPAD - _N)))
    xyz_t = pcs.reshape(3, _B, _CHUNKS, 128).transpose(0, 2, 1, 3)
    lin = (jnp.arange(_CHUNKS, dtype=jnp.int32)[:, None, None] * 128
           + jnp.arange(128, dtype=jnp.int32)[None, None, :]
           + jnp.zeros((1, _B, 1), jnp.int32)).astype(jnp.float32)
    return pl.pallas_call(
        _fps_body,
        out_shape=jax.ShapeDtypeStruct((_PALL, _B), jnp.float32),
        scratch_shapes=[pltpu.VMEM((_CHUNKS, _B, 128), jnp.float32)],
    )(xyz_t, lin)


_SC_NW = 32
_SC_ROWS = _T // _SC_NW
_SC_CHUNK = 128


def _sc_gather_body(tab_ref, idx_ref, out_ref, idx_v, rows_v, sem):
    wid = lax.axis_index("s") * 2 + lax.axis_index("c")
    base = wid * _SC_ROWS
    for j in range(_SC_ROWS // _SC_CHUNK):
        off = base + j * _SC_CHUNK
        pltpu.sync_copy(idx_ref.at[pl.ds(off, _SC_CHUNK)], idx_v)
        pltpu.async_copy(tab_ref.at[idx_v], rows_v, sem).wait()
        pltpu.sync_copy(rows_v, out_ref.at[pl.ds(off, _SC_CHUNK)])


def _run_sc_gather(table, flat_idx):
    mesh = plsc.VectorSubcoreMesh(core_axis_name="c", subcore_axis_name="s")
    k = functools.partial(
        pl.kernel,
        mesh=mesh,
        out_type=jax.ShapeDtypeStruct((_T, 16), jnp.float32),
        scratch_types=[
            pltpu.VMEM((_SC_CHUNK,), jnp.int32),
            pltpu.VMEM((_SC_CHUNK, 16), jnp.float32),
            pltpu.SemaphoreType.DMA,
        ],
        compiler_params=pltpu.CompilerParams(use_tc_tiling_on_sc=False),
    )(_sc_gather_body)
    return k(table, flat_idx)


def _layernorm(x):
    mu = jnp.mean(x, axis=-1, keepdims=True)
    xc = x - mu
    var = jnp.mean(xc * xc, axis=-1, keepdims=True)
    return xc * lax.rsqrt(var + 1e-5)


def _tf_body(feat_ref, pct_ref, win_ref, wqkv_ref, wo_ref,
             w1_ref, w2_ref, wout_ref, out_ref, c_ref, work_ref):
    feat = feat_ref[0]
    d2 = None
    for c in range(3):
        col = feat[:, 3 + c:4 + c]
        row = pct_ref[0, c:c + 1, :]
        dd = col - row
        dd = dd * dd
        d2 = dd if d2 is None else d2 + dd
    r2 = jnp.float32(_RADIUS ** 2)
    inrad = d2 <= r2
    inradf = inrad.astype(jnp.float32)
    count = jnp.sum(inradf, axis=1, keepdims=True)
    maxc = jnp.max(count)
    colidx_i = lax.broadcasted_iota(jnp.int32, (_NPT, _NPT), 1)
    rowidx_i = lax.broadcasted_iota(jnp.int32, (_NPT, _NPT), 0)
    colidx = colidx_i.astype(jnp.float32)
    diag = colidx_i == rowidx_i
    c_ref[...] = inradf + jnp.where(diag, jnp.float32(_MAXN) - count, 0.0)

    @pl.when(maxc > jnp.float32(_MAXN))
    def _rare():
        work_ref[...] = jnp.where(inrad, d2, jnp.inf)
        c_ref[...] = jnp.zeros((_NPT, _NPT), jnp.float32)
        selfcol = lax.broadcasted_iota(jnp.int32, (_NPT, 1), 0).astype(
            jnp.float32)

        def ex(j, carry):
            work = work_ref[...]
            m = jnp.min(work, axis=1, keepdims=True)
            am = jnp.min(jnp.where(work == m, colidx, 2e9), axis=1,
                         keepdims=True)
            isfin = m != jnp.inf
            chosen = jnp.where(isfin, am, selfcol)
            c_ref[...] = c_ref[...] + (colidx == chosen).astype(jnp.float32)
            work_ref[...] = jnp.where((colidx == am) & isfin, jnp.inf, work)
            return carry

        lax.fori_loop(0, _MAXN, ex, 0)

    x = jnp.dot(feat, win_ref[...], preferred_element_type=jnp.float32)
    logc = jnp.log(c_ref[...])
    inv_scale = jnp.float32(1.0 / 4.0)
    for l in range(_NL):
        xn = _layernorm(x)
        o_heads = []
        for h in range(_H):
            qkv = jnp.dot(xn, wqkv_ref[l, h],
                          preferred_element_type=jnp.float32)
            qh = qkv[:, 0:_DH]
            kh = qkv[:, _DH:2 * _DH]
            vh1 = jnp.concatenate(
                [qkv[:, 2 * _DH:3 * _DH],
                 jnp.ones((_NPT, 1), jnp.float32)], axis=1)
            logits = lax.dot_general(
                qh, kh, (((1,), (1,)), ((), ())),
                preferred_element_type=jnp.float32) * inv_scale
            neg = logits + logc
            mx = jnp.max(neg, axis=1, keepdims=True)
            e = jnp.exp(neg - mx)
            ov = jnp.dot(e, vh1, preferred_element_type=jnp.float32)
            oh = ov[:, :_DH] / ov[:, _DH:_DH + 1]
            o_heads.append(oh)
        o = jnp.concatenate(o_heads, axis=1)
        x = x + jnp.dot(o, wo_ref[l], preferred_element_type=jnp.float32)
        xn2 = _layernorm(x)
        hid = jnp.maximum(
            jnp.dot(xn2, w1_ref[l], preferred_element_type=jnp.float32), 0.0)
        x = x + jnp.dot(hid, w2_ref[l], preferred_element_type=jnp.float32)
    out = jnp.dot(_layernorm(x), wout_ref[...],
                  preferred_element_type=jnp.float32)
    out_ref[0] = out


def _run_transformer(feat, pct, win_p, wqkv, wo, w1, w2, wout):
    full = lambda *shape: pl.BlockSpec(shape, lambda b: (0,) * len(shape))
    return pl.pallas_call(
        _tf_body,
        grid=(_B,),
        in_specs=[
            pl.BlockSpec((1, _NPT, 8), lambda b: (b, 0, 0)),
            pl.BlockSpec((1, 3, _NPT), lambda b: (b, 0, 0)),
            full(8, _D),
            full(_NL, _H, _D, 3 * _DH),
            full(_NL, _D, _D),
            full(_NL, _D, _DFF),
            full(_NL, _DFF, _D),
            full(_D, _DOUT),
        ],
        out_specs=pl.BlockSpec((1, _NPT, _DOUT), lambda b: (b, 0, 0)),
        out_shape=jax.ShapeDtypeStruct((_B, _NPT, _DOUT), jnp.float32),
        scratch_shapes=[pltpu.VMEM((_NPT, _NPT), jnp.float32),
                        pltpu.VMEM((_NPT, _NPT), jnp.float32)],
    )(feat, pct, win_p, wqkv, wo, w1, w2, wout)


def kernel(point_clouds, cloud_colors, W_in, Wq, Wk, Wv, Wo, W1, W2, W_out):
    idxs_f = _run_fps(point_clouds)
    idxs = idxs_f.astype(jnp.int32)
    sel = jax.random.permutation(jax.random.key(1), _PALL)[:_NPT]
    fps_idx = idxs[sel, :].T
    flat_idx = (fps_idx
                + (jnp.arange(_B, dtype=jnp.int32) * _N)[:, None]
                ).reshape(_T)
    table = jnp.pad(jnp.concatenate([cloud_colors, point_clouds], axis=-1),
                    ((0, 0), (0, 0), (0, 10))).reshape(_B * _N, 16)
    rows = _run_sc_gather(table, flat_idx)
    feat = rows[:, :8].reshape(_B, _NPT, 8)
    pc = rows[:, 3:6].reshape(_B, _NPT, 3)
    pct = jnp.transpose(pc, (0, 2, 1))
    win_p = jnp.pad(W_in, ((0, 2), (0, 0)))
    wqkv = jnp.concatenate(
        [w.reshape(_NL, _D, _H, _DH).transpose(0, 2, 1, 3)
         for w in (Wq, Wk, Wv)], axis=-1)
    out = _run_transformer(feat, pct, win_p, wqkv, Wo, W1, W2, W_out)
    features = jnp.transpose(out, (0, 2, 1))
    return features, pc
